# Initial kernel scaffold; baseline (speedup 1.0000x reference)
import jax
import jax.numpy as jnp
from jax.experimental import pallas as pl
from jax.experimental.pallas import tpu as pltpu

_DIM = 1024
_INTER = 512
_E = 8
_TOPK = 2
_SHINTER = 1024


def _dot_t(a, b, prec=None):
    return jax.lax.dot_general(
        a, b, (((1,), (1,)), ((), ())),
        preferred_element_type=jnp.float32, precision=prec)


def _moe_body(x_ref, wg_ref, w1_ref, w2_ref, w3_ref,
              ws1_ref, bs1_ref, ws2_ref, bs2_ref, ws3_ref, bs3_ref,
              y_ref, l_ref, xb_scr, w_scr):
    e = pl.program_id(0)
    T = x_ref.shape[0]

    @pl.when(e == 0)
    def _gate():
        xb_scr[...] = x_ref[...].astype(jnp.bfloat16)
        scores = _dot_t(x_ref[...], wg_ref[...], jax.lax.Precision.HIGHEST)
        p = jax.nn.sigmoid(scores)
        iota = jax.lax.broadcasted_iota(jnp.int32, p.shape, 1)
        m1 = jnp.max(p, axis=1, keepdims=True)
        am1 = jnp.min(jnp.where(p == m1, iota, _E), axis=1, keepdims=True)
        p2 = jnp.where(iota == am1, -1.0, p)
        m2 = jnp.max(p2, axis=1, keepdims=True)
        am2 = jnp.min(jnp.where(p2 == m2, iota, _E), axis=1, keepdims=True)
        s = m1 + m2
        w = (jnp.where(iota == am1, m1, 0.0) +
             jnp.where(iota == am2, m2, 0.0)) / s
        w_scr[...] = w
        sel = ((iota == am1) | (iota == am2)).astype(jnp.float32)
        counts = jnp.sum(sel, axis=0, keepdims=True)
        probs = jnp.sum(w, axis=0, keepdims=True)
        f_i = _E * counts / (_TOPK * T)
        p_i = probs / T
        l_ref[...] = jnp.sum(f_i * p_i, axis=1, keepdims=True)

    @pl.when(e < _E)
    def _routed():
        xb = xb_scr[...]
        h1 = _dot_t(xb, w1_ref[0])
        h3 = _dot_t(xb, w3_ref[0])
        h = (jax.nn.silu(h1) * h3).astype(jnp.bfloat16)
        out = _dot_t(h, w2_ref[0])
        iota = jax.lax.broadcasted_iota(jnp.int32, (T, _E), 1)
        wtok = jnp.sum(jnp.where(iota == e, w_scr[...], 0.0),
                       axis=1, keepdims=True)
        contrib = out * wtok

        @pl.when(e == 0)
        def _():
            y_ref[...] = contrib

        @pl.when(e > 0)
        def _():
            y_ref[...] += contrib

    @pl.when(e == _E)
    def _shared():
        xb = xb_scr[...]
        g1 = _dot_t(xb, ws1_ref[...]) + bs1_ref[...]
        g3 = _dot_t(xb, ws3_ref[...]) + bs3_ref[...]
        hs = (jax.nn.silu(g1) * g3).astype(jnp.bfloat16)
        z = _dot_t(hs, ws2_ref[...]) + bs2_ref[...]
        y_ref[...] += z


def kernel(x, Wg, W1, W2, W3, Ws1, bs1, Ws2, bs2, Ws3, bs3):
    orig_shape = x.shape
    xf = x.reshape(-1, _DIM)
    T = xf.shape[0]
    bf = jnp.bfloat16
    W1b, W2b, W3b = W1.astype(bf), W2.astype(bf), W3.astype(bf)
    Ws1b, Ws2b, Ws3b = Ws1.astype(bf), Ws2.astype(bf), Ws3.astype(bf)
    bs1r = bs1.reshape(1, _SHINTER)
    bs2r = bs2.reshape(1, _DIM)
    bs3r = bs3.reshape(1, _SHINTER)

    const2 = lambda shape: pl.BlockSpec(shape, lambda e: (0, 0))
    expert3 = lambda shape: pl.BlockSpec(
        shape, lambda e: (jnp.minimum(e, _E - 1), 0, 0))

    y, l = pl.pallas_call(
        _moe_body,
        grid=(_E + 1,),
        in_specs=[
            const2((T, _DIM)),
            const2((_E, _DIM)),
            expert3((1, _INTER, _DIM)),
            expert3((1, _DIM, _INTER)),
            expert3((1, _INTER, _DIM)),
            const2((_SHINTER, _DIM)),
            const2((1, _SHINTER)),
            const2((_DIM, _SHINTER)),
            const2((1, _DIM)),
            const2((_SHINTER, _DIM)),
            const2((1, _SHINTER)),
        ],
        out_specs=[
            const2((T, _DIM)),
            const2((1, 1)),
        ],
        out_shape=[
            jax.ShapeDtypeStruct((T, _DIM), jnp.float32),
            jax.ShapeDtypeStruct((1, 1), jnp.float32),
        ],
        scratch_shapes=[
            pltpu.VMEM((T, _DIM), bf),
            pltpu.VMEM((T, _E), jnp.float32),
        ],
        compiler_params=pltpu.CompilerParams(
            dimension_semantics=("arbitrary",)),
    )(xf, Wg, W1b, W2b, W3b, Ws1b, bs1r, Ws2b, bs2r, Ws3b, bs3r)
    return y.reshape(orig_shape), l[0, 0]


# speedup vs baseline: 1.7978x; 1.7978x over previous
import jax
import jax.numpy as jnp
from jax.experimental import pallas as pl
from jax.experimental.pallas import tpu as pltpu

_DIM = 1024
_INTER = 512
_E = 8
_TOPK = 2
_SHINTER = 1024


def _dot_t(a, b, prec=None):
    return jax.lax.dot_general(
        a, b, (((1,), (1,)), ((), ())),
        preferred_element_type=jnp.float32, precision=prec)


def _moe_body(x_ref, wg_ref, w1_ref, w2_ref, w3_ref,
              ws1_ref, bs1_ref, ws2_ref, bs2_ref, ws3_ref, bs3_ref,
              y_ref, l_ref, xb_scr, w_scr):
    e = pl.program_id(0)
    T = x_ref.shape[0]

    @pl.when(e == 0)
    def _gate():
        xb_scr[...] = x_ref[...].astype(jnp.bfloat16)
        scores = _dot_t(x_ref[...], wg_ref[...])
        p = jax.nn.sigmoid(scores)
        iota = jax.lax.broadcasted_iota(jnp.int32, p.shape, 1)
        m1 = jnp.max(p, axis=1, keepdims=True)
        am1 = jnp.min(jnp.where(p == m1, iota, _E), axis=1, keepdims=True)
        p2 = jnp.where(iota == am1, -1.0, p)
        m2 = jnp.max(p2, axis=1, keepdims=True)
        am2 = jnp.min(jnp.where(p2 == m2, iota, _E), axis=1, keepdims=True)
        s = m1 + m2
        w = (jnp.where(iota == am1, m1, 0.0) +
             jnp.where(iota == am2, m2, 0.0)) / s
        w_scr[...] = w
        sel = ((iota == am1) | (iota == am2)).astype(jnp.float32)
        counts = jnp.sum(sel, axis=0, keepdims=True)
        probs = jnp.sum(w, axis=0, keepdims=True)
        f_i = _E * counts / (_TOPK * T)
        p_i = probs / T
        l_ref[...] = jnp.sum(f_i * p_i, axis=1, keepdims=True)

    @pl.when(e < _E)
    def _routed():
        xb = xb_scr[...]
        h1 = _dot_t(xb, w1_ref[0])
        h3 = _dot_t(xb, w3_ref[0])
        h = (jax.nn.silu(h1) * h3).astype(jnp.bfloat16)
        out = _dot_t(h, w2_ref[0])
        iota = jax.lax.broadcasted_iota(jnp.int32, (T, _E), 1)
        wtok = jnp.sum(jnp.where(iota == e, w_scr[...], 0.0),
                       axis=1, keepdims=True)
        contrib = out * wtok

        @pl.when(e == 0)
        def _():
            y_ref[...] = contrib

        @pl.when(e > 0)
        def _():
            y_ref[...] += contrib

    @pl.when(e == _E)
    def _shared():
        xb = xb_scr[...]
        g1 = _dot_t(xb, ws1_ref[...]) + bs1_ref[...]
        g3 = _dot_t(xb, ws3_ref[...]) + bs3_ref[...]
        hs = (jax.nn.silu(g1) * g3).astype(jnp.bfloat16)
        z = _dot_t(hs, ws2_ref[...]) + bs2_ref[...]
        y_ref[...] += z


def kernel(x, Wg, W1, W2, W3, Ws1, bs1, Ws2, bs2, Ws3, bs3):
    orig_shape = x.shape
    xf = x.reshape(-1, _DIM)
    T = xf.shape[0]
    bf = jnp.bfloat16
    W1b, W2b, W3b = W1.astype(bf), W2.astype(bf), W3.astype(bf)
    Ws1b, Ws2b, Ws3b = Ws1.astype(bf), Ws2.astype(bf), Ws3.astype(bf)
    bs1r = bs1.reshape(1, _SHINTER)
    bs2r = bs2.reshape(1, _DIM)
    bs3r = bs3.reshape(1, _SHINTER)

    const2 = lambda shape: pl.BlockSpec(shape, lambda e: (0, 0))
    expert3 = lambda shape: pl.BlockSpec(
        shape, lambda e: (jnp.minimum(e, _E - 1), 0, 0))

    y, l = pl.pallas_call(
        _moe_body,
        grid=(_E + 1,),
        in_specs=[
            const2((T, _DIM)),
            const2((_E, _DIM)),
            expert3((1, _INTER, _DIM)),
            expert3((1, _DIM, _INTER)),
            expert3((1, _INTER, _DIM)),
            const2((_SHINTER, _DIM)),
            const2((1, _SHINTER)),
            const2((_DIM, _SHINTER)),
            const2((1, _DIM)),
            const2((_SHINTER, _DIM)),
            const2((1, _SHINTER)),
        ],
        out_specs=[
            const2((T, _DIM)),
            const2((1, 1)),
        ],
        out_shape=[
            jax.ShapeDtypeStruct((T, _DIM), jnp.float32),
            jax.ShapeDtypeStruct((1, 1), jnp.float32),
        ],
        scratch_shapes=[
            pltpu.VMEM((T, _DIM), bf),
            pltpu.VMEM((T, _E), jnp.float32),
        ],
        compiler_params=pltpu.CompilerParams(
            dimension_semantics=("arbitrary",)),
    )(xf, Wg, W1b, W2b, W3b, Ws1b, bs1r, Ws2b, bs2r, Ws3b, bs3r)
    return y.reshape(orig_shape), l[0, 0]
